# baseline (speedup 1.0000x reference)
import functools

import jax
import jax.numpy as jnp
from jax import lax
from jax.experimental import pallas as pl
from jax.experimental.pallas import tpu as pltpu
from jax.experimental.pallas import tpu_sc as plsc

N = 4096
D = 128
K = 2048
RT = 512
NW = 32
RPW = K // NW
RB = 8
L = 16
f32 = jnp.float32


def _k1_body(feat_ref, w1_ref, b1_ref, adj_ref, h_ref, x1_ref):
    i = pl.program_id(0)

    @pl.when(i == 0)
    def _():
        x1_ref[...] = jnp.dot(feat_ref[...], w1_ref[...],
                              preferred_element_type=f32)

    acc = jnp.dot(adj_ref[...], x1_ref[...], preferred_element_type=f32)
    h_ref[...] = jnp.maximum(acc + b1_ref[...], 0.0)


def _k2_body(h_ref, w2_ref, b2_ref, ws8_ref, adj_ref, emb_ref, v8_ref, x2_ref):
    i = pl.program_id(0)

    @pl.when(i == 0)
    def _():
        x2_ref[...] = jnp.dot(h_ref[...], w2_ref[...],
                              preferred_element_type=f32)

    emb = jnp.dot(adj_ref[...], x2_ref[...], preferred_element_type=f32)
    emb = emb + b2_ref[...]
    emb_ref[...] = emb
    v8_ref[...] = jnp.dot(emb, ws8_ref[...], preferred_element_type=f32)


def _k3_body(v8_ref, bs_ref, adj_ref, y_ref):
    z8 = jnp.dot(adj_ref[...], v8_ref[...], preferred_element_type=f32)
    z = z8[:, :1] + bs_ref[...]
    y_ref[...] = 1.0 / (1.0 + jnp.exp(-z))


def _k4a_body(yc_ref, yr_ref, ranks_ref):
    i = pl.program_id(0)
    yc = yc_ref[...]
    yr = yr_ref[...]
    j_ids = i * RT + lax.broadcasted_iota(jnp.int32, (RT, 1), 0)
    i_ids = lax.broadcasted_iota(jnp.int32, (RT, N), 1)
    beats = (yc > yr) | ((yc == yr) & (j_ids < i_ids))
    contrib = jnp.sum(beats.astype(jnp.int32), axis=0, keepdims=True)

    @pl.when(i == 0)
    def _():
        ranks_ref[...] = contrib

    @pl.when(i > 0)
    def _():
        ranks_ref[...] = ranks_ref[...] + contrib


def _k4b_body(ranks_ref, yr_ref, idx_ref, mult_ref):
    i = pl.program_id(0)
    ranks = ranks_ref[...]
    yr = yr_ref[...]
    r_col = i * RT + lax.broadcasted_iota(jnp.int32, (RT, 1), 0)
    eq = ranks == r_col
    col_ids = lax.broadcasted_iota(jnp.int32, (RT, N), 1)
    idx_ref[...] = jnp.sum(jnp.where(eq, col_ids, 0), axis=1, keepdims=True)
    m_row = yr + (1.0 - yr)
    mult_ref[...] = jnp.sum(jnp.where(eq, m_row, 0.0), axis=1, keepdims=True)


def _tc_pipeline(features, adj, W1, b1, W2, b2, Ws, bs):
    b1r = b1.reshape(1, D)
    b2r = b2.reshape(1, D)
    bsr = bs.reshape(1, 1)
    ws8 = jnp.concatenate([Ws, jnp.zeros((D, 7), f32)], axis=1)

    h = pl.pallas_call(
        _k1_body,
        grid=(N // RT,),
        in_specs=[
            pl.BlockSpec((N, D), lambda i: (0, 0)),
            pl.BlockSpec((D, D), lambda i: (0, 0)),
            pl.BlockSpec((1, D), lambda i: (0, 0)),
            pl.BlockSpec((RT, N), lambda i: (i, 0)),
        ],
        out_specs=pl.BlockSpec((RT, D), lambda i: (i, 0)),
        out_shape=jax.ShapeDtypeStruct((N, D), f32),
        scratch_shapes=[pltpu.VMEM((N, D), f32)],
    )(features, W1, b1r, adj)

    emb, v8 = pl.pallas_call(
        _k2_body,
        grid=(N // RT,),
        in_specs=[
            pl.BlockSpec((N, D), lambda i: (0, 0)),
            pl.BlockSpec((D, D), lambda i: (0, 0)),
            pl.BlockSpec((1, D), lambda i: (0, 0)),
            pl.BlockSpec((D, 8), lambda i: (0, 0)),
            pl.BlockSpec((RT, N), lambda i: (i, 0)),
        ],
        out_specs=[
            pl.BlockSpec((RT, D), lambda i: (i, 0)),
            pl.BlockSpec((RT, 8), lambda i: (i, 0)),
        ],
        out_shape=[
            jax.ShapeDtypeStruct((N, D), f32),
            jax.ShapeDtypeStruct((N, 8), f32),
        ],
        scratch_shapes=[pltpu.VMEM((N, D), f32)],
    )(h, W2, b2r, ws8, adj)

    y = pl.pallas_call(
        _k3_body,
        grid=(N // RT,),
        in_specs=[
            pl.BlockSpec((N, 8), lambda i: (0, 0)),
            pl.BlockSpec((1, 1), lambda i: (0, 0)),
            pl.BlockSpec((RT, N), lambda i: (i, 0)),
        ],
        out_specs=pl.BlockSpec((RT, 1), lambda i: (i, 0)),
        out_shape=jax.ShapeDtypeStruct((N, 1), f32),
    )(v8, bsr, adj)

    yr = y.reshape(1, N)

    ranks = pl.pallas_call(
        _k4a_body,
        grid=(N // RT,),
        in_specs=[
            pl.BlockSpec((RT, 1), lambda i: (i, 0)),
            pl.BlockSpec((1, N), lambda i: (0, 0)),
        ],
        out_specs=pl.BlockSpec((1, N), lambda i: (0, 0)),
        out_shape=jax.ShapeDtypeStruct((1, N), jnp.int32),
    )(y, yr)

    idx2, mult2 = pl.pallas_call(
        _k4b_body,
        grid=(K // RT,),
        in_specs=[
            pl.BlockSpec((1, N), lambda i: (0, 0)),
            pl.BlockSpec((1, N), lambda i: (0, 0)),
        ],
        out_specs=[
            pl.BlockSpec((RT, 1), lambda i: (i, 0)),
            pl.BlockSpec((RT, 1), lambda i: (i, 0)),
        ],
        out_shape=[
            jax.ShapeDtypeStruct((K, 1), jnp.int32),
            jax.ShapeDtypeStruct((K, 1), f32),
        ],
    )(ranks, yr)

    return emb, idx2.reshape(K), mult2.reshape(K)


def _sc_body(adj_hbm, emb_hbm, idx_hbm, mult_hbm, out_emb, out_adj,
             idx_all, my_idx, my_mult, emb_rows, row_blk, out_blk, sem):
    c = lax.axis_index("c")
    s = lax.axis_index("s")
    wid = s * 2 + c
    base = wid * RPW

    pltpu.sync_copy(idx_hbm, idx_all)
    pltpu.sync_copy(idx_hbm.at[pl.ds(base, RPW)], my_idx)
    pltpu.sync_copy(mult_hbm.at[pl.ds(base, RPW)], my_mult)

    pltpu.async_copy(emb_hbm.at[my_idx], emb_rows, sem).wait()

    def _scale_row(r, carry):
        m = plsc.load_gather(my_mult, [jnp.full((L,), r, jnp.int32)])
        for ch in range(D // L):
            sl = pl.ds(ch * L, L)
            emb_rows[r, sl] = emb_rows[r, sl] * m
        return carry

    lax.fori_loop(0, RPW, _scale_row, 0)
    pltpu.sync_copy(emb_rows, out_emb.at[pl.ds(base, RPW)])

    def _blk(b, carry):
        pltpu.async_copy(adj_hbm.at[my_idx.at[pl.ds(b * RB, RB)]],
                         row_blk, sem).wait()

        def _row(rb, carry2):
            rsplat = jnp.full((L,), rb, jnp.int32)

            def _col(cc, carry3):
                cols = idx_all[pl.ds(cc * L, L)]
                out_blk[rb, pl.ds(cc * L, L)] = plsc.load_gather(
                    row_blk, [rsplat, cols])
                return carry3

            lax.fori_loop(0, K // L, _col, 0)
            return carry2

        lax.fori_loop(0, RB, _row, 0)
        pltpu.sync_copy(out_blk, out_adj.at[pl.ds(base + b * RB, RB)])
        return carry

    lax.fori_loop(0, RPW // RB, _blk, 0)


def _sc_gather(adj, emb, idx, mult):
    call = pl.kernel(
        _sc_body,
        out_type=[
            jax.ShapeDtypeStruct((K, D), f32),
            jax.ShapeDtypeStruct((K, K), f32),
        ],
        mesh=plsc.VectorSubcoreMesh(core_axis_name="c", subcore_axis_name="s"),
        scratch_types=[
            pltpu.VMEM((K,), jnp.int32),
            pltpu.VMEM((RPW,), jnp.int32),
            pltpu.VMEM((RPW,), f32),
            pltpu.VMEM((RPW, D), f32),
            pltpu.VMEM((RB, N), f32),
            pltpu.VMEM((RB, K), f32),
            pltpu.SemaphoreType.DMA,
        ],
        compiler_params=pltpu.CompilerParams(needs_layout_passes=False),
    )
    return call(adj, emb, idx, mult)


@jax.jit
def kernel(features, adj, W1, b1, W2, b2, Ws, bs):
    emb, idx, mult = _tc_pipeline(features, adj, W1, b1, W2, b2, Ws, bs)
    out_emb, out_adj = _sc_gather(adj, emb, idx, mult)
    return out_emb, out_adj


# speedup vs baseline: 1.3609x; 1.3609x over previous
import functools

import jax
import jax.numpy as jnp
from jax import lax
from jax.experimental import pallas as pl
from jax.experimental.pallas import tpu as pltpu
from jax.experimental.pallas import tpu_sc as plsc

N = 4096
D = 128
K = 2048
RT = 512
NW = 32
RPW = K // NW
RB = 8
L = 16
f32 = jnp.float32


def _k1_body(feat_ref, w1_ref, b1_ref, adj_ref, h_ref, x1_ref):
    i = pl.program_id(0)

    @pl.when(i == 0)
    def _():
        x1_ref[...] = jnp.dot(feat_ref[...], w1_ref[...],
                              preferred_element_type=f32)

    acc = jnp.dot(adj_ref[...], x1_ref[...], preferred_element_type=f32)
    h_ref[...] = jnp.maximum(acc + b1_ref[...], 0.0)


def _k2_body(h_ref, w2_ref, b2_ref, ws8_ref, adj_ref, emb_ref, v8_ref, x2_ref):
    i = pl.program_id(0)

    @pl.when(i == 0)
    def _():
        x2_ref[...] = jnp.dot(h_ref[...], w2_ref[...],
                              preferred_element_type=f32)

    emb = jnp.dot(adj_ref[...], x2_ref[...], preferred_element_type=f32)
    emb = emb + b2_ref[...]
    emb_ref[...] = emb
    v8_ref[...] = jnp.dot(emb, ws8_ref[...], preferred_element_type=f32)


def _k3_body(v8_ref, bs_ref, adj_ref, y_ref):
    z8 = jnp.dot(adj_ref[...], v8_ref[...], preferred_element_type=f32)
    z = z8[:, :1] + bs_ref[...]
    y_ref[...] = 1.0 / (1.0 + jnp.exp(-z))


def _k4a_body(yc_ref, yr_ref, ranks_ref):
    i = pl.program_id(0)
    yc = yc_ref[...]
    yr = yr_ref[...]
    j_ids = i * RT + lax.broadcasted_iota(jnp.int32, (RT, 1), 0)
    i_ids = lax.broadcasted_iota(jnp.int32, (RT, N), 1)
    beats = (yc > yr) | ((yc == yr) & (j_ids < i_ids))
    contrib = jnp.sum(beats.astype(jnp.int32), axis=0, keepdims=True)

    @pl.when(i == 0)
    def _():
        ranks_ref[...] = contrib

    @pl.when(i > 0)
    def _():
        ranks_ref[...] = ranks_ref[...] + contrib


def _k4b_body(ranks_ref, yr_ref, idx_ref, mult_ref):
    i = pl.program_id(0)
    ranks = ranks_ref[...]
    yr = yr_ref[...]
    r_col = i * RT + lax.broadcasted_iota(jnp.int32, (RT, 1), 0)
    eq = ranks == r_col
    col_ids = lax.broadcasted_iota(jnp.int32, (RT, N), 1)
    idx_ref[...] = jnp.sum(jnp.where(eq, col_ids, 0), axis=1, keepdims=True)
    m_row = yr + (1.0 - yr)
    mult_ref[...] = jnp.sum(jnp.where(eq, m_row, 0.0), axis=1, keepdims=True)


def _tc_pipeline(features, adj, W1, b1, W2, b2, Ws, bs):
    b1r = b1.reshape(1, D)
    b2r = b2.reshape(1, D)
    bsr = bs.reshape(1, 1)
    ws8 = jnp.concatenate([Ws, jnp.zeros((D, 7), f32)], axis=1)

    h = pl.pallas_call(
        _k1_body,
        grid=(N // RT,),
        in_specs=[
            pl.BlockSpec((N, D), lambda i: (0, 0)),
            pl.BlockSpec((D, D), lambda i: (0, 0)),
            pl.BlockSpec((1, D), lambda i: (0, 0)),
            pl.BlockSpec((RT, N), lambda i: (i, 0)),
        ],
        out_specs=pl.BlockSpec((RT, D), lambda i: (i, 0)),
        out_shape=jax.ShapeDtypeStruct((N, D), f32),
        scratch_shapes=[pltpu.VMEM((N, D), f32)],
    )(features, W1, b1r, adj)

    emb, v8 = pl.pallas_call(
        _k2_body,
        grid=(N // RT,),
        in_specs=[
            pl.BlockSpec((N, D), lambda i: (0, 0)),
            pl.BlockSpec((D, D), lambda i: (0, 0)),
            pl.BlockSpec((1, D), lambda i: (0, 0)),
            pl.BlockSpec((D, 8), lambda i: (0, 0)),
            pl.BlockSpec((RT, N), lambda i: (i, 0)),
        ],
        out_specs=[
            pl.BlockSpec((RT, D), lambda i: (i, 0)),
            pl.BlockSpec((RT, 8), lambda i: (i, 0)),
        ],
        out_shape=[
            jax.ShapeDtypeStruct((N, D), f32),
            jax.ShapeDtypeStruct((N, 8), f32),
        ],
        scratch_shapes=[pltpu.VMEM((N, D), f32)],
    )(h, W2, b2r, ws8, adj)

    y = pl.pallas_call(
        _k3_body,
        grid=(N // RT,),
        in_specs=[
            pl.BlockSpec((N, 8), lambda i: (0, 0)),
            pl.BlockSpec((1, 1), lambda i: (0, 0)),
            pl.BlockSpec((RT, N), lambda i: (i, 0)),
        ],
        out_specs=pl.BlockSpec((RT, 1), lambda i: (i, 0)),
        out_shape=jax.ShapeDtypeStruct((N, 1), f32),
    )(v8, bsr, adj)

    yr = y.reshape(1, N)

    ranks = pl.pallas_call(
        _k4a_body,
        grid=(N // RT,),
        in_specs=[
            pl.BlockSpec((RT, 1), lambda i: (i, 0)),
            pl.BlockSpec((1, N), lambda i: (0, 0)),
        ],
        out_specs=pl.BlockSpec((1, N), lambda i: (0, 0)),
        out_shape=jax.ShapeDtypeStruct((1, N), jnp.int32),
    )(y, yr)

    idx2, mult2 = pl.pallas_call(
        _k4b_body,
        grid=(K // RT,),
        in_specs=[
            pl.BlockSpec((1, N), lambda i: (0, 0)),
            pl.BlockSpec((1, N), lambda i: (0, 0)),
        ],
        out_specs=[
            pl.BlockSpec((RT, 1), lambda i: (i, 0)),
            pl.BlockSpec((RT, 1), lambda i: (i, 0)),
        ],
        out_shape=[
            jax.ShapeDtypeStruct((K, 1), jnp.int32),
            jax.ShapeDtypeStruct((K, 1), f32),
        ],
    )(ranks, yr)

    return emb, idx2.reshape(K), mult2.reshape(K)


def _sc_body(adj_hbm, emb_hbm, idx_hbm, mult_hbm, out_emb, out_adj,
             idx_all, my_idx, my_mult, emb_rows,
             row_a, row_b, out_a, out_b,
             sem, sem_ra, sem_rb, sem_oa, sem_ob):
    c = lax.axis_index("c")
    s = lax.axis_index("s")
    wid = s * 2 + c
    base = wid * RPW

    pltpu.sync_copy(idx_hbm, idx_all)
    pltpu.sync_copy(idx_hbm.at[pl.ds(base, RPW)], my_idx)
    pltpu.sync_copy(mult_hbm.at[pl.ds(base, RPW)], my_mult)

    pltpu.async_copy(emb_hbm.at[my_idx], emb_rows, sem).wait()

    def _scale_row(r, carry):
        m = plsc.load_gather(my_mult, [jnp.full((L,), r, jnp.int32)])
        for ch in range(D // L):
            sl = pl.ds(ch * L, L)
            emb_rows[r, sl] = emb_rows[r, sl] * m
        return carry

    lax.fori_loop(0, RPW, _scale_row, 0)
    pltpu.sync_copy(emb_rows, out_emb.at[pl.ds(base, RPW)])

    nblk = RPW // RB
    rows = (row_a, row_b)
    rsems = (sem_ra, sem_rb)
    outs = (out_a, out_b)
    osems = (sem_oa, sem_ob)
    splats = [jnp.full((L,), rb, jnp.int32) for rb in range(RB)]

    in_cp = [None, None]
    out_cp = [None, None]
    in_cp[0] = pltpu.async_copy(adj_hbm.at[my_idx.at[pl.ds(0, RB)]],
                                row_a, sem_ra)
    for b in range(nblk):
        k = b % 2
        if b + 1 < nblk:
            in_cp[1 - k] = pltpu.async_copy(
                adj_hbm.at[my_idx.at[pl.ds((b + 1) * RB, RB)]],
                rows[1 - k], rsems[1 - k])
        in_cp[k].wait()
        if b >= 2:
            out_cp[k].wait()
        row_blk = rows[k]
        out_blk = outs[k]

        def _col(cc, carry):
            cols = idx_all[pl.ds(cc * L, L)]
            sl = pl.ds(cc * L, L)
            for rb in range(RB):
                out_blk[rb, sl] = plsc.load_gather(row_blk, [splats[rb], cols])
            return carry

        lax.fori_loop(0, K // L, _col, 0)
        out_cp[k] = pltpu.async_copy(out_blk,
                                     out_adj.at[pl.ds(base + b * RB, RB)],
                                     osems[k])
    out_cp[0].wait()
    out_cp[1].wait()


def _sc_gather(adj, emb, idx, mult):
    call = pl.kernel(
        _sc_body,
        out_type=[
            jax.ShapeDtypeStruct((K, D), f32),
            jax.ShapeDtypeStruct((K, K), f32),
        ],
        mesh=plsc.VectorSubcoreMesh(core_axis_name="c", subcore_axis_name="s"),
        scratch_types=[
            pltpu.VMEM((K,), jnp.int32),
            pltpu.VMEM((RPW,), jnp.int32),
            pltpu.VMEM((RPW,), f32),
            pltpu.VMEM((RPW, D), f32),
            pltpu.VMEM((RB, N), f32),
            pltpu.VMEM((RB, N), f32),
            pltpu.VMEM((RB, K), f32),
            pltpu.VMEM((RB, K), f32),
            pltpu.SemaphoreType.DMA,
            pltpu.SemaphoreType.DMA,
            pltpu.SemaphoreType.DMA,
            pltpu.SemaphoreType.DMA,
            pltpu.SemaphoreType.DMA,
        ],
        compiler_params=pltpu.CompilerParams(needs_layout_passes=False),
    )
    return call(adj, emb, idx, mult)


@jax.jit
def kernel(features, adj, W1, b1, W2, b2, Ws, bs):
    emb, idx, mult = _tc_pipeline(features, adj, W1, b1, W2, b2, Ws, bs)
    out_emb, out_adj = _sc_gather(adj, emb, idx, mult)
    return out_emb, out_adj


# speedup vs baseline: 1.6508x; 1.2130x over previous
import functools

import jax
import jax.numpy as jnp
from jax import lax
from jax.experimental import pallas as pl
from jax.experimental.pallas import tpu as pltpu
from jax.experimental.pallas import tpu_sc as plsc

N = 4096
D = 128
K = 2048
RT = 512
NW = 32
RPW = K // NW
RB = 8
L = 16
f32 = jnp.float32


def _k123_body(feat_ref, w1_ref, b1_ref, w2_ref, b2_ref, ws8_ref, bs_ref,
               adj_ref, emb_ref, y_ref, x1_ref, h_ref, x2_ref, v8_ref):
    i = pl.program_id(0)

    @pl.when(i == 0)
    def _():
        x1_ref[...] = jnp.dot(feat_ref[...], w1_ref[...],
                              preferred_element_type=f32)

    @pl.when(i < 8)
    def _():
        acc = jnp.dot(adj_ref[...], x1_ref[...], preferred_element_type=f32)
        h_ref[pl.ds(i * RT, RT), :] = jnp.maximum(acc + b1_ref[...], 0.0)

    @pl.when(i == 8)
    def _():
        x2_ref[...] = jnp.dot(h_ref[...], w2_ref[...],
                              preferred_element_type=f32)

    @pl.when((i >= 8) & (i < 16))
    def _():
        j = i - 8
        emb = jnp.dot(adj_ref[...], x2_ref[...], preferred_element_type=f32)
        emb = emb + b2_ref[...]
        emb_ref[...] = emb
        v8_ref[pl.ds(j * RT, RT), :] = jnp.dot(emb, ws8_ref[...],
                                               preferred_element_type=f32)

    @pl.when(i >= 16)
    def _():
        z8 = jnp.dot(adj_ref[...], v8_ref[...], preferred_element_type=f32)
        z = z8[:, :1] + bs_ref[...]
        y_ref[...] = 1.0 / (1.0 + jnp.exp(-z))


def _k4_body(yc_ref, yr_ref, idx_ref, mult_ref, ranks_ref):
    i = pl.program_id(0)
    yr = yr_ref[...]

    @pl.when(i < 8)
    def _():
        yc = yc_ref[...]
        j_ids = i * RT + lax.broadcasted_iota(jnp.int32, (RT, 1), 0)
        i_ids = lax.broadcasted_iota(jnp.int32, (RT, N), 1)
        beats = (yc > yr) | ((yc == yr) & (j_ids < i_ids))
        contrib = jnp.sum(beats.astype(jnp.int32), axis=0, keepdims=True)

        @pl.when(i == 0)
        def _():
            ranks_ref[...] = contrib

        @pl.when(i > 0)
        def _():
            ranks_ref[...] = ranks_ref[...] + contrib

    @pl.when(i >= 8)
    def _():
        ranks = ranks_ref[...]
        r_col = (i - 8) * RT + lax.broadcasted_iota(jnp.int32, (RT, 1), 0)
        eq = ranks == r_col
        col_ids = lax.broadcasted_iota(jnp.int32, (RT, N), 1)
        idx_ref[...] = jnp.sum(jnp.where(eq, col_ids, 0), axis=1,
                               keepdims=True)
        m_row = yr + (1.0 - yr)
        mult_ref[...] = jnp.sum(jnp.where(eq, m_row, 0.0), axis=1,
                                keepdims=True)


def _tc_pipeline(features, adj, W1, b1, W2, b2, Ws, bs):
    b1r = b1.reshape(1, D)
    b2r = b2.reshape(1, D)
    bsr = bs.reshape(1, 1)
    ws8 = jnp.concatenate([Ws, jnp.zeros((D, 7), f32)], axis=1)
    const = lambda i: (0, 0)

    emb, y = pl.pallas_call(
        _k123_body,
        grid=(24,),
        in_specs=[
            pl.BlockSpec((N, D), const),
            pl.BlockSpec((D, D), const),
            pl.BlockSpec((1, D), const),
            pl.BlockSpec((D, D), const),
            pl.BlockSpec((1, D), const),
            pl.BlockSpec((D, 8), const),
            pl.BlockSpec((1, 1), const),
            pl.BlockSpec((RT, N), lambda i: (i % 8, 0)),
        ],
        out_specs=[
            pl.BlockSpec((RT, D), lambda i: (jnp.clip(i - 8, 0, 7), 0)),
            pl.BlockSpec((RT, 1), lambda i: (jnp.clip(i - 16, 0, 7), 0)),
        ],
        out_shape=[
            jax.ShapeDtypeStruct((N, D), f32),
            jax.ShapeDtypeStruct((N, 1), f32),
        ],
        scratch_shapes=[
            pltpu.VMEM((N, D), f32),
            pltpu.VMEM((N, D), f32),
            pltpu.VMEM((N, D), f32),
            pltpu.VMEM((N, 8), f32),
        ],
    )(features, W1, b1r, W2, b2r, ws8, bsr, adj)

    yr = y.reshape(1, N)

    idx2, mult2 = pl.pallas_call(
        _k4_body,
        grid=(12,),
        in_specs=[
            pl.BlockSpec((RT, 1), lambda i: (jnp.minimum(i, 7), 0)),
            pl.BlockSpec((1, N), const),
        ],
        out_specs=[
            pl.BlockSpec((RT, 1), lambda i: (jnp.clip(i - 8, 0, 3), 0)),
            pl.BlockSpec((RT, 1), lambda i: (jnp.clip(i - 8, 0, 3), 0)),
        ],
        out_shape=[
            jax.ShapeDtypeStruct((K, 1), jnp.int32),
            jax.ShapeDtypeStruct((K, 1), f32),
        ],
        scratch_shapes=[pltpu.VMEM((1, N), jnp.int32)],
    )(y, yr)

    return emb, idx2.reshape(K), mult2.reshape(K)


def _sc_body(adj_hbm, emb_hbm, idx_hbm, mult_hbm, out_emb, out_adj,
             idx_all, my_idx, my_mult, emb_rows,
             row_a, row_b, out_a, out_b,
             sem, sem_ra, sem_rb, sem_oa, sem_ob):
    c = lax.axis_index("c")
    s = lax.axis_index("s")
    wid = s * 2 + c
    base = wid * RPW

    pltpu.sync_copy(idx_hbm, idx_all)
    pltpu.sync_copy(idx_hbm.at[pl.ds(base, RPW)], my_idx)
    pltpu.sync_copy(mult_hbm.at[pl.ds(base, RPW)], my_mult)

    pltpu.async_copy(emb_hbm.at[my_idx], emb_rows, sem).wait()

    def _scale_row(r, carry):
        m = plsc.load_gather(my_mult, [jnp.full((L,), r, jnp.int32)])
        for ch in range(D // L):
            sl = pl.ds(ch * L, L)
            emb_rows[r, sl] = emb_rows[r, sl] * m
        return carry

    lax.fori_loop(0, RPW, _scale_row, 0)
    pltpu.sync_copy(emb_rows, out_emb.at[pl.ds(base, RPW)])

    nblk = RPW // RB
    rows = (row_a, row_b)
    rsems = (sem_ra, sem_rb)
    outs = (out_a, out_b)
    osems = (sem_oa, sem_ob)
    splats = [jnp.full((L,), rb, jnp.int32) for rb in range(RB)]

    in_cp = [None, None]
    out_cp = [None, None]
    in_cp[0] = pltpu.async_copy(adj_hbm.at[my_idx.at[pl.ds(0, RB)]],
                                row_a, sem_ra)
    for b in range(nblk):
        k = b % 2
        if b + 1 < nblk:
            in_cp[1 - k] = pltpu.async_copy(
                adj_hbm.at[my_idx.at[pl.ds((b + 1) * RB, RB)]],
                rows[1 - k], rsems[1 - k])
        in_cp[k].wait()
        if b >= 2:
            out_cp[k].wait()
        row_blk = rows[k]
        out_blk = outs[k]

        @plsc.parallel_loop(0, K // L, 1, unroll=4)
        def _col(cc):
            cols = idx_all[pl.ds(cc * L, L)]
            sl = pl.ds(cc * L, L)
            for rb in range(RB):
                out_blk[rb, sl] = plsc.load_gather(row_blk, [splats[rb], cols])
        out_cp[k] = pltpu.async_copy(out_blk,
                                     out_adj.at[pl.ds(base + b * RB, RB)],
                                     osems[k])
    out_cp[0].wait()
    out_cp[1].wait()


def _sc_gather(adj, emb, idx, mult):
    call = pl.kernel(
        _sc_body,
        out_type=[
            jax.ShapeDtypeStruct((K, D), f32),
            jax.ShapeDtypeStruct((K, K), f32),
        ],
        mesh=plsc.VectorSubcoreMesh(core_axis_name="c", subcore_axis_name="s"),
        scratch_types=[
            pltpu.VMEM((K,), jnp.int32),
            pltpu.VMEM((RPW,), jnp.int32),
            pltpu.VMEM((RPW,), f32),
            pltpu.VMEM((RPW, D), f32),
            pltpu.VMEM((RB, N), f32),
            pltpu.VMEM((RB, N), f32),
            pltpu.VMEM((RB, K), f32),
            pltpu.VMEM((RB, K), f32),
            pltpu.SemaphoreType.DMA,
            pltpu.SemaphoreType.DMA,
            pltpu.SemaphoreType.DMA,
            pltpu.SemaphoreType.DMA,
            pltpu.SemaphoreType.DMA,
        ],
        compiler_params=pltpu.CompilerParams(needs_layout_passes=False),
    )
    return call(adj, emb, idx, mult)


@jax.jit
def kernel(features, adj, W1, b1, W2, b2, Ws, bs):
    emb, idx, mult = _tc_pipeline(features, adj, W1, b1, W2, b2, Ws, bs)
    out_emb, out_adj = _sc_gather(adj, emb, idx, mult)
    return out_emb, out_adj


# speedup vs baseline: 1.7099x; 1.0358x over previous
import functools

import jax
import jax.numpy as jnp
from jax import lax
from jax.experimental import pallas as pl
from jax.experimental.pallas import tpu as pltpu
from jax.experimental.pallas import tpu_sc as plsc

N = 4096
D = 128
K = 2048
RT = 512
NW = 32
RPW = K // NW
RB = 8
L = 16
f32 = jnp.float32


def _tc_body(feat_ref, w1_ref, b1_ref, w2_ref, b2_ref, ws_ref, bs_ref,
             adj_ref, emb_ref, idx_ref, mult_ref,
             x1_ref, h_ref, x2_ref, v8_ref, yrow_ref, ranks_ref):
    i = pl.program_id(0)

    @pl.when(i == 0)
    def _():
        x1_ref[...] = jnp.dot(feat_ref[...], w1_ref[...],
                              preferred_element_type=f32)

    @pl.when(i < 8)
    def _():
        acc = jnp.dot(adj_ref[...], x1_ref[...], preferred_element_type=f32)
        h_ref[pl.ds(i * RT, RT), :] = jnp.maximum(acc + b1_ref[...], 0.0)

    @pl.when(i == 8)
    def _():
        x2_ref[...] = jnp.dot(h_ref[...], w2_ref[...],
                              preferred_element_type=f32)

    @pl.when((i >= 8) & (i < 16))
    def _():
        j = i - 8
        emb = jnp.dot(adj_ref[...], x2_ref[...], preferred_element_type=f32)
        emb = emb + b2_ref[...]
        emb_ref[...] = emb
        v8 = jnp.dot(emb, ws_ref[...], preferred_element_type=f32)
        v8_ref[pl.ds(j * RT, RT), :] = jnp.broadcast_to(v8, (RT, 8))

    @pl.when((i >= 16) & (i < 24))
    def _():
        j = i - 16
        z8 = jnp.dot(adj_ref[...], v8_ref[...], preferred_element_type=f32)
        z = z8[:, :1] + bs_ref[...]
        y = 1.0 / (1.0 + jnp.exp(-z))
        yrow_ref[:, pl.ds(j * RT, RT)] = jnp.transpose(y, (1, 0))

    @pl.when((i >= 24) & (i < 32))
    def _():
        j = i - 24
        yr = yrow_ref[...]
        yc = jnp.transpose(yrow_ref[:, pl.ds(j * RT, RT)], (1, 0))
        j_ids = j * RT + lax.broadcasted_iota(jnp.int32, (RT, 1), 0)
        i_ids = lax.broadcasted_iota(jnp.int32, (RT, N), 1)
        beats = (yc > yr) | ((yc == yr) & (j_ids < i_ids))
        contrib = jnp.sum(beats.astype(jnp.int32), axis=0, keepdims=True)

        @pl.when(j == 0)
        def _():
            ranks_ref[...] = contrib

        @pl.when(j > 0)
        def _():
            ranks_ref[...] = ranks_ref[...] + contrib

    @pl.when(i >= 32)
    def _():
        yr = yrow_ref[...]
        ranks = ranks_ref[...]
        r_col = (i - 32) * RT + lax.broadcasted_iota(jnp.int32, (RT, 1), 0)
        eq = ranks == r_col
        col_ids = lax.broadcasted_iota(jnp.int32, (RT, N), 1)
        idx_ref[...] = jnp.sum(jnp.where(eq, col_ids, 0), axis=1)
        m_row = yr + (1.0 - yr)
        mult_ref[...] = jnp.sum(jnp.where(eq, m_row, 0.0), axis=1)


def _tc_pipeline(features, adj, W1, b1, W2, b2, Ws, bs):
    b1r = b1.reshape(1, D)
    b2r = b2.reshape(1, D)
    bsr = bs.reshape(1, 1)
    const = lambda i: (0, 0)

    emb, idx, mult = pl.pallas_call(
        _tc_body,
        grid=(36,),
        in_specs=[
            pl.BlockSpec((N, D), const),
            pl.BlockSpec((D, D), const),
            pl.BlockSpec((1, D), const),
            pl.BlockSpec((D, D), const),
            pl.BlockSpec((1, D), const),
            pl.BlockSpec((D, 1), const),
            pl.BlockSpec((1, 1), const),
            pl.BlockSpec((RT, N), lambda i: (jnp.where(i < 24, i % 8, 7), 0)),
        ],
        out_specs=[
            pl.BlockSpec((RT, D), lambda i: (jnp.clip(i - 8, 0, 7), 0)),
            pl.BlockSpec((RT,), lambda i: (jnp.clip(i - 32, 0, 3),)),
            pl.BlockSpec((RT,), lambda i: (jnp.clip(i - 32, 0, 3),)),
        ],
        out_shape=[
            jax.ShapeDtypeStruct((N, D), f32),
            jax.ShapeDtypeStruct((K,), jnp.int32),
            jax.ShapeDtypeStruct((K,), f32),
        ],
        scratch_shapes=[
            pltpu.VMEM((N, D), f32),
            pltpu.VMEM((N, D), f32),
            pltpu.VMEM((N, D), f32),
            pltpu.VMEM((N, 8), f32),
            pltpu.VMEM((1, N), f32),
            pltpu.VMEM((1, N), jnp.int32),
        ],
    )(features, W1, b1r, W2, b2r, Ws, bsr, adj)

    return emb, idx, mult


def _sc_body(adj_hbm, emb_hbm, idx_hbm, mult_hbm, out_emb, out_adj,
             idx_all, my_idx, my_mult, emb_rows,
             row_a, row_b, out_a, out_b,
             sem, sem_ra, sem_rb, sem_oa, sem_ob):
    c = lax.axis_index("c")
    s = lax.axis_index("s")
    wid = s * 2 + c
    base = wid * RPW

    pltpu.sync_copy(idx_hbm, idx_all)
    pltpu.sync_copy(idx_hbm.at[pl.ds(base, RPW)], my_idx)
    pltpu.sync_copy(mult_hbm.at[pl.ds(base, RPW)], my_mult)

    pltpu.async_copy(emb_hbm.at[my_idx], emb_rows, sem).wait()

    def _scale_row(r, carry):
        m = plsc.load_gather(my_mult, [jnp.full((L,), r, jnp.int32)])
        for ch in range(D // L):
            sl = pl.ds(ch * L, L)
            emb_rows[r, sl] = emb_rows[r, sl] * m
        return carry

    lax.fori_loop(0, RPW, _scale_row, 0)
    pltpu.sync_copy(emb_rows, out_emb.at[pl.ds(base, RPW)])

    nblk = RPW // RB
    rows = (row_a, row_b)
    rsems = (sem_ra, sem_rb)
    outs = (out_a, out_b)
    osems = (sem_oa, sem_ob)
    splats = [jnp.full((L,), rb, jnp.int32) for rb in range(RB)]

    in_cp = [None, None]
    out_cp = [None, None]
    in_cp[0] = pltpu.async_copy(adj_hbm.at[my_idx.at[pl.ds(0, RB)]],
                                row_a, sem_ra)
    for b in range(nblk):
        k = b % 2
        if b + 1 < nblk:
            in_cp[1 - k] = pltpu.async_copy(
                adj_hbm.at[my_idx.at[pl.ds((b + 1) * RB, RB)]],
                rows[1 - k], rsems[1 - k])
        in_cp[k].wait()
        if b >= 2:
            out_cp[k].wait()
        row_blk = rows[k]
        out_blk = outs[k]

        @plsc.parallel_loop(0, K // L, 1, unroll=4)
        def _col(cc):
            cols = idx_all[pl.ds(cc * L, L)]
            sl = pl.ds(cc * L, L)
            for rb in range(RB):
                out_blk[rb, sl] = plsc.load_gather(row_blk, [splats[rb], cols])
        out_cp[k] = pltpu.async_copy(out_blk,
                                     out_adj.at[pl.ds(base + b * RB, RB)],
                                     osems[k])
    out_cp[0].wait()
    out_cp[1].wait()


def _sc_gather(adj, emb, idx, mult):
    call = pl.kernel(
        _sc_body,
        out_type=[
            jax.ShapeDtypeStruct((K, D), f32),
            jax.ShapeDtypeStruct((K, K), f32),
        ],
        mesh=plsc.VectorSubcoreMesh(core_axis_name="c", subcore_axis_name="s"),
        scratch_types=[
            pltpu.VMEM((K,), jnp.int32),
            pltpu.VMEM((RPW,), jnp.int32),
            pltpu.VMEM((RPW,), f32),
            pltpu.VMEM((RPW, D), f32),
            pltpu.VMEM((RB, N), f32),
            pltpu.VMEM((RB, N), f32),
            pltpu.VMEM((RB, K), f32),
            pltpu.VMEM((RB, K), f32),
            pltpu.SemaphoreType.DMA,
            pltpu.SemaphoreType.DMA,
            pltpu.SemaphoreType.DMA,
            pltpu.SemaphoreType.DMA,
            pltpu.SemaphoreType.DMA,
        ],
        compiler_params=pltpu.CompilerParams(needs_layout_passes=False),
    )
    return call(adj, emb, idx, mult)


@jax.jit
def kernel(features, adj, W1, b1, W2, b2, Ws, bs):
    emb, idx, mult = _tc_pipeline(features, adj, W1, b1, W2, b2, Ws, bs)
    out_emb, out_adj = _sc_gather(adj, emb, idx, mult)
    return out_emb, out_adj


# speedup vs baseline: 1.7870x; 1.0451x over previous
import functools

import jax
import jax.numpy as jnp
from jax import lax
from jax.experimental import pallas as pl
from jax.experimental.pallas import tpu as pltpu
from jax.experimental.pallas import tpu_sc as plsc

N = 4096
D = 128
K = 2048
RT = 512
NW = 32
RPW = K // NW
RB = 8
L = 16
f32 = jnp.float32


CB = 4


def _p23_j(t):
    return jnp.where(t == 0, 7, jnp.where(t < 4, t + 3, t - 4))


def _tc_body(feat_ref, w1_ref, b1_ref, w2_ref, b2_ref, ws_ref, bs_ref,
             adj_ref, emb_ref, idx_ref, mult_ref,
             adjv_ref, x1_ref, x2_ref, v8_ref, yrow_ref, ranks_ref):
    i = pl.program_id(0)

    @pl.when(i == 0)
    def _():
        x1_ref[...] = jnp.dot(feat_ref[...], w1_ref[...],
                              preferred_element_type=f32)

    @pl.when(i < 8)
    def _():
        adj_blk = adj_ref[...]
        acc = jnp.dot(adj_blk, x1_ref[...], preferred_element_type=f32)
        h_tile = jnp.maximum(acc + b1_ref[...], 0.0)
        x2_ref[pl.ds(i * RT, RT), :] = jnp.dot(h_tile, w2_ref[...],
                                               preferred_element_type=f32)

        @pl.when(i < CB)
        def _():
            adjv_ref[pl.ds(i * RT, RT), :] = adj_blk

    @pl.when((i >= 8) & (i < 16))
    def _():
        j = _p23_j(i - 8)

        def _emit_emb(adj_blk):
            emb = jnp.dot(adj_blk, x2_ref[...], preferred_element_type=f32)
            emb = emb + b2_ref[...]
            emb_ref[...] = emb
            v8 = jnp.dot(emb, ws_ref[...], preferred_element_type=f32)
            v8_ref[:, pl.ds(j * RT, RT)] = jnp.broadcast_to(
                jnp.transpose(v8, (1, 0)), (8, RT))

        @pl.when(j < CB)
        def _():
            _emit_emb(adjv_ref[pl.ds(j * RT, RT), :])

        @pl.when(j >= CB)
        def _():
            _emit_emb(adj_ref[...])

    @pl.when((i >= 16) & (i < 24))
    def _():
        j = _p23_j(i - 16)

        def _emit_y(adj_blk):
            z8 = lax.dot_general(adj_blk, v8_ref[...],
                                 (((1,), (1,)), ((), ())),
                                 preferred_element_type=f32)
            z = z8[:, :1] + bs_ref[...]
            y = 1.0 / (1.0 + jnp.exp(-z))
            yrow_ref[:, pl.ds(j * RT, RT)] = jnp.transpose(y, (1, 0))

        @pl.when(j < CB)
        def _():
            _emit_y(adjv_ref[pl.ds(j * RT, RT), :])

        @pl.when(j >= CB)
        def _():
            _emit_y(adj_ref[...])

    @pl.when((i >= 24) & (i < 32))
    def _():
        j = i - 24
        yr = yrow_ref[...]
        yc = jnp.transpose(yrow_ref[:, pl.ds(j * RT, RT)], (1, 0))
        j_ids = j * RT + lax.broadcasted_iota(jnp.int32, (RT, 1), 0)
        i_ids = lax.broadcasted_iota(jnp.int32, (RT, N), 1)
        beats = (yc > yr) | ((yc == yr) & (j_ids < i_ids))
        contrib = jnp.sum(beats.astype(jnp.int32), axis=0, keepdims=True)

        @pl.when(j == 0)
        def _():
            ranks_ref[...] = contrib

        @pl.when(j > 0)
        def _():
            ranks_ref[...] = ranks_ref[...] + contrib

    @pl.when(i >= 32)
    def _():
        yr = yrow_ref[...]
        ranks = ranks_ref[...]
        r_col = (i - 32) * RT + lax.broadcasted_iota(jnp.int32, (RT, 1), 0)
        eq = ranks == r_col
        col_ids = lax.broadcasted_iota(jnp.int32, (RT, N), 1)
        idx_ref[...] = jnp.sum(jnp.where(eq, col_ids, 0), axis=1)
        m_row = yr + (1.0 - yr)
        mult_ref[...] = jnp.sum(jnp.where(eq, m_row, 0.0), axis=1)


def _adj_map(i):
    t = jnp.where(i < 16, i - 8, jnp.where(i < 24, i - 16, 8))
    p23 = jnp.where(t == 0, 7, jnp.where(t < 4, t + 3, 6))
    return (jnp.where(i < 8, i, p23), 0)


def _emb_map(i):
    return (jnp.where(i <= 8, 7,
                      jnp.where(i < 12, i - 5,
                                jnp.where(i < 16, i - 12, 3))), 0)


def _tc_pipeline(features, adj, W1, b1, W2, b2, Ws, bs):
    b1r = b1.reshape(1, D)
    b2r = b2.reshape(1, D)
    bsr = bs.reshape(1, 1)
    const = lambda i: (0, 0)

    emb, idx, mult = pl.pallas_call(
        _tc_body,
        grid=(36,),
        in_specs=[
            pl.BlockSpec((N, D), const),
            pl.BlockSpec((D, D), const),
            pl.BlockSpec((1, D), const),
            pl.BlockSpec((D, D), const),
            pl.BlockSpec((1, D), const),
            pl.BlockSpec((D, 1), const),
            pl.BlockSpec((1, 1), const),
            pl.BlockSpec((RT, N), _adj_map),
        ],
        out_specs=[
            pl.BlockSpec((RT, D), _emb_map),
            pl.BlockSpec((RT,), lambda i: (jnp.clip(i - 32, 0, 3),)),
            pl.BlockSpec((RT,), lambda i: (jnp.clip(i - 32, 0, 3),)),
        ],
        out_shape=[
            jax.ShapeDtypeStruct((N, D), f32),
            jax.ShapeDtypeStruct((K,), jnp.int32),
            jax.ShapeDtypeStruct((K,), f32),
        ],
        scratch_shapes=[
            pltpu.VMEM((CB * RT, N), f32),
            pltpu.VMEM((N, D), f32),
            pltpu.VMEM((N, D), f32),
            pltpu.VMEM((8, N), f32),
            pltpu.VMEM((1, N), f32),
            pltpu.VMEM((1, N), jnp.int32),
        ],
        compiler_params=pltpu.CompilerParams(
            vmem_limit_bytes=128 * 1024 * 1024),
    )(features, W1, b1r, W2, b2r, Ws, bsr, adj)

    return emb, idx, mult


def _sc_body(adj_hbm, emb_hbm, idx_hbm, mult_hbm, out_emb, out_adj,
             idx_all, my_idx, my_mult, emb_rows,
             row_a, row_b, out_a, out_b,
             sem, sem_ra, sem_rb, sem_oa, sem_ob):
    c = lax.axis_index("c")
    s = lax.axis_index("s")
    wid = s * 2 + c
    base = wid * RPW

    pltpu.sync_copy(idx_hbm, idx_all)
    pltpu.sync_copy(idx_hbm.at[pl.ds(base, RPW)], my_idx)
    pltpu.sync_copy(mult_hbm.at[pl.ds(base, RPW)], my_mult)

    pltpu.async_copy(emb_hbm.at[my_idx], emb_rows, sem).wait()

    def _scale_row(r, carry):
        m = plsc.load_gather(my_mult, [jnp.full((L,), r, jnp.int32)])
        for ch in range(D // L):
            sl = pl.ds(ch * L, L)
            emb_rows[r, sl] = emb_rows[r, sl] * m
        return carry

    lax.fori_loop(0, RPW, _scale_row, 0)
    pltpu.sync_copy(emb_rows, out_emb.at[pl.ds(base, RPW)])

    nblk = RPW // RB
    rows = (row_a, row_b)
    rsems = (sem_ra, sem_rb)
    outs = (out_a, out_b)
    osems = (sem_oa, sem_ob)
    splats = [jnp.full((L,), rb, jnp.int32) for rb in range(RB)]

    in_cp = [None, None]
    out_cp = [None, None]
    in_cp[0] = pltpu.async_copy(adj_hbm.at[my_idx.at[pl.ds(0, RB)]],
                                row_a, sem_ra)
    for b in range(nblk):
        k = b % 2
        if b + 1 < nblk:
            in_cp[1 - k] = pltpu.async_copy(
                adj_hbm.at[my_idx.at[pl.ds((b + 1) * RB, RB)]],
                rows[1 - k], rsems[1 - k])
        in_cp[k].wait()
        if b >= 2:
            out_cp[k].wait()
        row_blk = rows[k]
        out_blk = outs[k]

        @plsc.parallel_loop(0, K // L, 1, unroll=4)
        def _col(cc):
            cols = idx_all[pl.ds(cc * L, L)]
            sl = pl.ds(cc * L, L)
            for rb in range(RB):
                out_blk[rb, sl] = plsc.load_gather(row_blk, [splats[rb], cols])
        out_cp[k] = pltpu.async_copy(out_blk,
                                     out_adj.at[pl.ds(base + b * RB, RB)],
                                     osems[k])
    out_cp[0].wait()
    out_cp[1].wait()


def _sc_gather(adj, emb, idx, mult):
    call = pl.kernel(
        _sc_body,
        out_type=[
            jax.ShapeDtypeStruct((K, D), f32),
            jax.ShapeDtypeStruct((K, K), f32),
        ],
        mesh=plsc.VectorSubcoreMesh(core_axis_name="c", subcore_axis_name="s"),
        scratch_types=[
            pltpu.VMEM((K,), jnp.int32),
            pltpu.VMEM((RPW,), jnp.int32),
            pltpu.VMEM((RPW,), f32),
            pltpu.VMEM((RPW, D), f32),
            pltpu.VMEM((RB, N), f32),
            pltpu.VMEM((RB, N), f32),
            pltpu.VMEM((RB, K), f32),
            pltpu.VMEM((RB, K), f32),
            pltpu.SemaphoreType.DMA,
            pltpu.SemaphoreType.DMA,
            pltpu.SemaphoreType.DMA,
            pltpu.SemaphoreType.DMA,
            pltpu.SemaphoreType.DMA,
        ],
        compiler_params=pltpu.CompilerParams(needs_layout_passes=False),
    )
    return call(adj, emb, idx, mult)


@jax.jit
def kernel(features, adj, W1, b1, W2, b2, Ws, bs):
    emb, idx, mult = _tc_pipeline(features, adj, W1, b1, W2, b2, Ws, bs)
    out_emb, out_adj = _sc_gather(adj, emb, idx, mult)
    return out_emb, out_adj
